# Initial kernel scaffold; baseline (speedup 1.0000x reference)
import jax
import jax.numpy as jnp
from jax.experimental import pallas as pl


def kernel(x, x_mask, y, ob, item_emb, W0, b0, W1, b1, W2, b2, W3, b3):
    raise NotImplementedError("write your pallas kernel here")


# speedup vs baseline: 12.1741x; 12.1741x over previous
import functools

import jax
import jax.numpy as jnp
from jax import lax
from jax.experimental import pallas as pl
from jax.experimental.pallas import tpu as pltpu
from jax.experimental.pallas import tpu_sc as plsc

B, L, D, V = 4096, 200, 64, 100000
ATTRS = (2, 4, 6, 11)
NATTR = 23

NC, NS = 2, 16
NW = NC * NS
BPW = B // NW
CH = 40
NCH = L // CH

NPAD = 128
BB = 512


def _sc_pool_body(x_hbm, emb_hbm, out_hbm, idx_v, rows_v, outs_v, sem0, sem1):
    wid = lax.axis_index("s") * NC + lax.axis_index("c")
    base = wid * BPW
    pltpu.sync_copy(x_hbm.at[pl.ds(base, BPW)], idx_v)

    sems = (sem0, sem1)

    def fire(j, buf):
        for c in range(NCH):
            pltpu.async_copy(
                emb_hbm.at[idx_v.at[j, c]],
                rows_v.at[buf, pl.ds(c * CH, CH)],
                sems[buf],
            )

    def drain(buf):
        pltpu.make_async_copy(
            emb_hbm.at[pl.ds(0, L)], rows_v.at[buf], sems[buf]
        ).wait()

    def bag_sum(buf):
        def rbody(r, accs):
            return tuple(
                a + rows_v[buf, r, pl.ds(16 * d, 16)]
                for d, a in enumerate(accs)
            )
        z = jnp.zeros((16,), jnp.float32)
        return lax.fori_loop(0, L, rbody, (z, z, z, z))

    def store(j, accs):
        for d, a in enumerate(accs):
            outs_v[j, pl.ds(16 * d, 16)] = a

    fire(0, 0)

    def pair(jj, carry):
        j0 = 2 * jj
        fire(j0 + 1, 1)
        drain(0)
        store(j0, bag_sum(0))

        @pl.when(jj < BPW // 2 - 1)
        def _():
            fire(j0 + 2, 0)

        drain(1)
        store(j0 + 1, bag_sum(1))
        return carry

    lax.fori_loop(0, BPW // 2, pair, 0)
    pltpu.sync_copy(outs_v, out_hbm.at[pl.ds(base, BPW)])


@jax.jit
def _sc_pool(x_r, item_emb):
    mesh = plsc.VectorSubcoreMesh(core_axis_name="c", subcore_axis_name="s")
    f = functools.partial(
        pl.kernel,
        out_type=jax.ShapeDtypeStruct((B, D), jnp.float32),
        mesh=mesh,
        scratch_types=[
            pltpu.VMEM((BPW, NCH, CH), jnp.int32),
            pltpu.VMEM((2, L, D), jnp.float32),
            pltpu.VMEM((BPW, D), jnp.float32),
            pltpu.SemaphoreType.DMA,
            pltpu.SemaphoreType.DMA,
        ],
        compiler_params=pltpu.CompilerParams(use_tc_tiling_on_sc=False),
    )(_sc_pool_body)
    return f(x_r, item_emb)


def _head_body(us_ref, xm_ref, y_ref, ob_ref, wt_ref, b_ref,
               logit_ref, loss_ref):
    i = pl.program_id(0)
    xlen = jnp.sum(xm_ref[...], axis=1, keepdims=True)
    rep = us_ref[...] / xlen
    A = jnp.dot(rep, wt_ref[...],
                preferred_element_type=jnp.float32) + b_ref[...]
    lane = lax.broadcasted_iota(jnp.int32, A.shape, 1)
    yob = y_ref[...] * ob_ref[...]
    logit = jnp.zeros_like(A)
    lossacc = jnp.float32(0.0)
    s = 0
    for t in ATTRS:
        e = s + t
        m = (lane >= s) & (lane < e)
        mx = jnp.max(jnp.where(m, A, -jnp.inf), axis=1, keepdims=True)
        ex = jnp.where(m, jnp.exp(A - mx), 0.0)
        ssum = jnp.sum(ex, axis=1, keepdims=True)
        logit = logit + ex / ssum
        logp = (A - mx) - jnp.log(ssum)
        lossacc = lossacc - jnp.sum(jnp.where(m, yob * logp, 0.0))
        s = e

    logit_ref[...] = logit

    @pl.when(i == 0)
    def _():
        loss_ref[...] = jnp.zeros((1, 1), jnp.float32)

    loss_ref[...] += jnp.full((1, 1), lossacc / B, jnp.float32)


@jax.jit
def _head(user_sum, x_mask, yp, obp, wt, bp):
    return pl.pallas_call(
        _head_body,
        grid=(B // BB,),
        in_specs=[
            pl.BlockSpec((BB, D), lambda i: (i, 0)),
            pl.BlockSpec((BB, L), lambda i: (i, 0)),
            pl.BlockSpec((BB, NPAD), lambda i: (i, 0)),
            pl.BlockSpec((BB, NPAD), lambda i: (i, 0)),
            pl.BlockSpec((D, NPAD), lambda i: (0, 0)),
            pl.BlockSpec((1, NPAD), lambda i: (0, 0)),
        ],
        out_specs=[
            pl.BlockSpec((BB, NPAD), lambda i: (i, 0)),
            pl.BlockSpec((1, 1), lambda i: (0, 0)),
        ],
        out_shape=[
            jax.ShapeDtypeStruct((B, NPAD), jnp.float32),
            jax.ShapeDtypeStruct((1, 1), jnp.float32),
        ],
    )(user_sum, x_mask, yp, obp, wt, bp)


def kernel(x, x_mask, y, ob, item_emb, W0, b0, W1, b1, W2, b2, W3, b3):
    x_r = x.reshape(B, NCH, CH)
    user_sum = _sc_pool(x_r, item_emb)

    wcat = jnp.concatenate([W0, W1, W2, W3], axis=0)
    bcat = jnp.concatenate([b0, b1, b2, b3], axis=0)
    wt = jnp.zeros((D, NPAD), jnp.float32).at[:, :NATTR].set(wcat.T)
    bp = jnp.zeros((1, NPAD), jnp.float32).at[0, :NATTR].set(bcat)
    yp = jnp.zeros((B, NPAD), jnp.float32).at[:, :NATTR].set(y)
    obp = jnp.zeros((B, NPAD), jnp.float32).at[:, :NATTR].set(ob)

    logit_p, loss_arr = _head(user_sum, x_mask, yp, obp, wt, bp)
    return logit_p[:, :NATTR], loss_arr[0, 0]


# speedup vs baseline: 13.5951x; 1.1167x over previous
import functools

import jax
import jax.numpy as jnp
from jax import lax
from jax.experimental import pallas as pl
from jax.experimental.pallas import tpu as pltpu
from jax.experimental.pallas import tpu_sc as plsc

B, L, D, V = 4096, 200, 64, 100000
ATTRS = (2, 4, 6, 11)
NATTR = 23

NC, NS = 2, 16
NW = NC * NS
BPW = B // NW
CHUNKS = ((0, 104), (104, 96))
UNROLL = 8

NPAD = 128
BB = 512


def _sc_pool_body(x_hbm, emb_hbm, out_hbm, idx_v, rows_v, outs_v, sem0, sem1):
    wid = lax.axis_index("s") * NC + lax.axis_index("c")
    base = wid * BPW
    pltpu.sync_copy(x_hbm.at[pl.ds(base, BPW)], idx_v)

    sems = (sem0, sem1)

    def fire(j, buf):
        for off, ln in CHUNKS:
            pltpu.async_copy(
                emb_hbm.at[idx_v.at[j, pl.ds(off, ln)]],
                rows_v.at[buf, pl.ds(off, ln)],
                sems[buf],
            )

    def drain(buf):
        pltpu.make_async_copy(
            emb_hbm.at[pl.ds(0, L)], rows_v.at[buf], sems[buf]
        ).wait()

    def bag_sum(buf):
        def rbody(i, accs):
            r0 = i * UNROLL
            for k in range(UNROLL):
                accs = tuple(
                    a + rows_v[buf, r0 + k, pl.ds(16 * d, 16)]
                    for d, a in enumerate(accs)
                )
            return accs
        z = jnp.zeros((16,), jnp.float32)
        return lax.fori_loop(0, L // UNROLL, rbody, (z, z, z, z))

    def store(j, accs):
        for d, a in enumerate(accs):
            outs_v[j, pl.ds(16 * d, 16)] = a

    fire(0, 0)

    def pair(jj, carry):
        j0 = 2 * jj
        fire(j0 + 1, 1)
        drain(0)
        store(j0, bag_sum(0))

        @pl.when(jj < BPW // 2 - 1)
        def _():
            fire(j0 + 2, 0)

        drain(1)
        store(j0 + 1, bag_sum(1))
        return carry

    lax.fori_loop(0, BPW // 2, pair, 0)
    pltpu.sync_copy(outs_v, out_hbm.at[pl.ds(base, BPW)])


@jax.jit
def _sc_pool(x_r, item_emb):
    mesh = plsc.VectorSubcoreMesh(core_axis_name="c", subcore_axis_name="s")
    f = functools.partial(
        pl.kernel,
        out_type=jax.ShapeDtypeStruct((B, D), jnp.float32),
        mesh=mesh,
        scratch_types=[
            pltpu.VMEM((BPW, L), jnp.int32),
            pltpu.VMEM((2, L, D), jnp.float32),
            pltpu.VMEM((BPW, D), jnp.float32),
            pltpu.SemaphoreType.DMA,
            pltpu.SemaphoreType.DMA,
        ],
        compiler_params=pltpu.CompilerParams(use_tc_tiling_on_sc=False),
    )(_sc_pool_body)
    return f(x_r, item_emb)


def _head_body(us_ref, xm_ref, y_ref, ob_ref, wt_ref, b_ref,
               logit_ref, loss_ref):
    i = pl.program_id(0)
    xlen = jnp.sum(xm_ref[...], axis=1, keepdims=True)
    rep = us_ref[...] / xlen
    A = jnp.dot(rep, wt_ref[...],
                preferred_element_type=jnp.float32) + b_ref[...]
    lane = lax.broadcasted_iota(jnp.int32, A.shape, 1)
    yob = y_ref[...] * ob_ref[...]
    logit = jnp.zeros_like(A)
    lossacc = jnp.float32(0.0)
    s = 0
    for t in ATTRS:
        e = s + t
        m = (lane >= s) & (lane < e)
        mx = jnp.max(jnp.where(m, A, -jnp.inf), axis=1, keepdims=True)
        ex = jnp.where(m, jnp.exp(A - mx), 0.0)
        ssum = jnp.sum(ex, axis=1, keepdims=True)
        logit = logit + ex / ssum
        logp = (A - mx) - jnp.log(ssum)
        lossacc = lossacc - jnp.sum(jnp.where(m, yob * logp, 0.0))
        s = e

    logit_ref[...] = logit

    @pl.when(i == 0)
    def _():
        loss_ref[...] = jnp.zeros((1, 1), jnp.float32)

    loss_ref[...] += jnp.full((1, 1), lossacc / B, jnp.float32)


@jax.jit
def _head(user_sum, x_mask, yp, obp, wt, bp):
    return pl.pallas_call(
        _head_body,
        grid=(B // BB,),
        in_specs=[
            pl.BlockSpec((BB, D), lambda i: (i, 0)),
            pl.BlockSpec((BB, L), lambda i: (i, 0)),
            pl.BlockSpec((BB, NPAD), lambda i: (i, 0)),
            pl.BlockSpec((BB, NPAD), lambda i: (i, 0)),
            pl.BlockSpec((D, NPAD), lambda i: (0, 0)),
            pl.BlockSpec((1, NPAD), lambda i: (0, 0)),
        ],
        out_specs=[
            pl.BlockSpec((BB, NPAD), lambda i: (i, 0)),
            pl.BlockSpec((1, 1), lambda i: (0, 0)),
        ],
        out_shape=[
            jax.ShapeDtypeStruct((B, NPAD), jnp.float32),
            jax.ShapeDtypeStruct((1, 1), jnp.float32),
        ],
    )(user_sum, x_mask, yp, obp, wt, bp)


def kernel(x, x_mask, y, ob, item_emb, W0, b0, W1, b1, W2, b2, W3, b3):
    user_sum = _sc_pool(x, item_emb)

    wcat = jnp.concatenate([W0, W1, W2, W3], axis=0)
    bcat = jnp.concatenate([b0, b1, b2, b3], axis=0)
    wt = jnp.zeros((D, NPAD), jnp.float32).at[:, :NATTR].set(wcat.T)
    bp = jnp.zeros((1, NPAD), jnp.float32).at[0, :NATTR].set(bcat)
    yp = jnp.zeros((B, NPAD), jnp.float32).at[:, :NATTR].set(y)
    obp = jnp.zeros((B, NPAD), jnp.float32).at[:, :NATTR].set(ob)

    logit_p, loss_arr = _head(user_sum, x_mask, yp, obp, wt, bp)
    return logit_p[:, :NATTR], loss_arr[0, 0]


# speedup vs baseline: 13.6485x; 1.0039x over previous
import functools

import jax
import jax.numpy as jnp
from jax import lax
from jax.experimental import pallas as pl
from jax.experimental.pallas import tpu as pltpu
from jax.experimental.pallas import tpu_sc as plsc

B, L, D, V = 4096, 200, 64, 100000
ATTRS = (2, 4, 6, 11)
NATTR = 23

NC, NS = 2, 16
NW = NC * NS
BPW = B // NW
CHUNKS = ((0, 104), (104, 96))
UNROLL = 8

NPAD = 128
BB = 512


def _sc_pool_body(x_hbm, emb_hbm, out_hbm, idx_v, rows_v, outs_v, sem0, sem1):
    wid = lax.axis_index("s") * NC + lax.axis_index("c")
    base = wid * BPW
    pltpu.sync_copy(x_hbm.at[pl.ds(base * L, BPW * L)], idx_v)

    sems = (sem0, sem1)

    def fire(j, buf):
        for off, ln in CHUNKS:
            pltpu.async_copy(
                emb_hbm.at[idx_v.at[pl.ds(j * L + off, ln)]],
                rows_v.at[buf, pl.ds(off, ln)],
                sems[buf],
            )

    def drain(buf):
        pltpu.make_async_copy(
            emb_hbm.at[pl.ds(0, L)], rows_v.at[buf], sems[buf]
        ).wait()

    def bag_sum(buf):
        def rbody(i, accs):
            r0 = i * UNROLL
            for k in range(UNROLL):
                accs = tuple(
                    a + rows_v[buf, r0 + k, pl.ds(16 * d, 16)]
                    for d, a in enumerate(accs)
                )
            return accs
        z = jnp.zeros((16,), jnp.float32)
        return lax.fori_loop(0, L // UNROLL, rbody, (z, z, z, z))

    def store(j, accs):
        for d, a in enumerate(accs):
            outs_v[j, pl.ds(16 * d, 16)] = a

    fire(0, 0)

    def pair(jj, carry):
        j0 = 2 * jj
        fire(j0 + 1, 1)
        drain(0)
        store(j0, bag_sum(0))

        @pl.when(jj < BPW // 2 - 1)
        def _():
            fire(j0 + 2, 0)

        drain(1)
        store(j0 + 1, bag_sum(1))
        return carry

    lax.fori_loop(0, BPW // 2, pair, 0)
    pltpu.sync_copy(outs_v, out_hbm.at[pl.ds(base, BPW)])


@jax.jit
def _sc_pool(x_r, item_emb):
    mesh = plsc.VectorSubcoreMesh(core_axis_name="c", subcore_axis_name="s")
    f = functools.partial(
        pl.kernel,
        out_type=jax.ShapeDtypeStruct((B, D), jnp.float32),
        mesh=mesh,
        scratch_types=[
            pltpu.VMEM((BPW * L,), jnp.int32),
            pltpu.VMEM((2, L, D), jnp.float32),
            pltpu.VMEM((BPW, D), jnp.float32),
            pltpu.SemaphoreType.DMA,
            pltpu.SemaphoreType.DMA,
        ],
        compiler_params=pltpu.CompilerParams(use_tc_tiling_on_sc=False),
    )(_sc_pool_body)
    return f(x_r, item_emb)


def _head_body(us_ref, xm_ref, y_ref, ob_ref, wt_ref, b_ref,
               logit_ref, loss_ref):
    i = pl.program_id(0)
    xlen = jnp.sum(xm_ref[...], axis=1, keepdims=True)
    rep = us_ref[...] / xlen
    A = jnp.dot(rep, wt_ref[...],
                preferred_element_type=jnp.float32) + b_ref[...]
    lane = lax.broadcasted_iota(jnp.int32, A.shape, 1)
    yob = y_ref[...] * ob_ref[...]
    logit = jnp.zeros_like(A)
    lossacc = jnp.float32(0.0)
    s = 0
    for t in ATTRS:
        e = s + t
        m = (lane >= s) & (lane < e)
        mx = jnp.max(jnp.where(m, A, -jnp.inf), axis=1, keepdims=True)
        ex = jnp.where(m, jnp.exp(A - mx), 0.0)
        ssum = jnp.sum(ex, axis=1, keepdims=True)
        logit = logit + ex / ssum
        logp = (A - mx) - jnp.log(ssum)
        lossacc = lossacc - jnp.sum(jnp.where(m, yob * logp, 0.0))
        s = e

    logit_ref[...] = logit

    @pl.when(i == 0)
    def _():
        loss_ref[...] = jnp.zeros((1, 1), jnp.float32)

    loss_ref[...] += jnp.full((1, 1), lossacc / B, jnp.float32)


@jax.jit
def _head(user_sum, x_mask, yp, obp, wt, bp):
    return pl.pallas_call(
        _head_body,
        grid=(B // BB,),
        in_specs=[
            pl.BlockSpec((BB, D), lambda i: (i, 0)),
            pl.BlockSpec((BB, L), lambda i: (i, 0)),
            pl.BlockSpec((BB, NPAD), lambda i: (i, 0)),
            pl.BlockSpec((BB, NPAD), lambda i: (i, 0)),
            pl.BlockSpec((D, NPAD), lambda i: (0, 0)),
            pl.BlockSpec((1, NPAD), lambda i: (0, 0)),
        ],
        out_specs=[
            pl.BlockSpec((BB, NPAD), lambda i: (i, 0)),
            pl.BlockSpec((1, 1), lambda i: (0, 0)),
        ],
        out_shape=[
            jax.ShapeDtypeStruct((B, NPAD), jnp.float32),
            jax.ShapeDtypeStruct((1, 1), jnp.float32),
        ],
    )(user_sum, x_mask, yp, obp, wt, bp)


def kernel(x, x_mask, y, ob, item_emb, W0, b0, W1, b1, W2, b2, W3, b3):
    user_sum = _sc_pool(x.reshape(-1), item_emb)

    wcat = jnp.concatenate([W0, W1, W2, W3], axis=0)
    bcat = jnp.concatenate([b0, b1, b2, b3], axis=0)
    wt = jnp.zeros((D, NPAD), jnp.float32).at[:, :NATTR].set(wcat.T)
    bp = jnp.zeros((1, NPAD), jnp.float32).at[0, :NATTR].set(bcat)
    yp = jnp.zeros((B, NPAD), jnp.float32).at[:, :NATTR].set(y)
    obp = jnp.zeros((B, NPAD), jnp.float32).at[:, :NATTR].set(ob)

    logit_p, loss_arr = _head(user_sum, x_mask, yp, obp, wt, bp)
    return logit_p[:, :NATTR], loss_arr[0, 0]
